# baseline (speedup 1.0000x reference)
import jax
import jax.numpy as jnp
from jax import lax
from jax.experimental import pallas as pl
from jax.experimental.pallas import tpu as pltpu

N = 50000
D = 256
S = 512
R = 512
NB = (N + R - 1) // R


def _body(x_ref, b3_ref, w_ref, bias_ref, out_ref, m_ref, z_ref):
    i = pl.program_id(0)

    @pl.when(i == 0)
    def _init():
        m_ref[0] = -jnp.inf
        z_ref[0] = 0.0

    xb = x_ref[...]
    s = jnp.dot(xb, w_ref[...], preferred_element_type=jnp.float32)
    s = s + bias_ref[0, 0]
    rows = i * R + lax.broadcasted_iota(jnp.int32, (R, 1), 0)
    valid = rows < N
    s = jnp.where(valid, s, -jnp.inf)

    m_old = m_ref[0]
    m_new = jnp.maximum(m_old, jnp.max(s))
    alpha = jnp.exp(m_old - m_new)
    p = jnp.exp(s - m_new)
    z_ref[0] = z_ref[0] * alpha + jnp.sum(p)
    m_ref[0] = m_new

    seg = b3_ref[0, 0, :]
    onehot = (lax.broadcasted_iota(jnp.int32, (S, R), 0) == seg[None, :])
    xp = jnp.where(valid, xb * p, 0.0)
    contrib = jnp.dot(onehot.astype(jnp.bfloat16), xp.astype(jnp.bfloat16),
                      preferred_element_type=jnp.float32)

    @pl.when(i == 0)
    def _first():
        out_ref[...] = contrib

    @pl.when(i > 0)
    def _acc():
        out_ref[...] = out_ref[...] * alpha + contrib

    @pl.when(i == NB - 1)
    def _fin():
        out_ref[...] = out_ref[...] / z_ref[0]


def kernel(x, batch, W, b):
    batch = batch.astype(jnp.int32)
    bpad = jnp.pad(batch, (0, NB * R - N))
    b3 = bpad.reshape(NB, 1, R)
    return pl.pallas_call(
        _body,
        grid=(NB,),
        in_specs=[
            pl.BlockSpec((R, D), lambda i: (i, 0)),
            pl.BlockSpec((1, 1, R), lambda i: (i, 0, 0)),
            pl.BlockSpec((D, 1), lambda i: (0, 0)),
            pl.BlockSpec((1, 1), lambda i: (0, 0)),
        ],
        out_specs=pl.BlockSpec((S, D), lambda i: (0, 0)),
        out_shape=jax.ShapeDtypeStruct((S, D), jnp.float32),
        scratch_shapes=[pltpu.SMEM((1,), jnp.float32),
                        pltpu.SMEM((1,), jnp.float32)],
    )(x, b3, W, b.reshape(1, 1))


# speedup vs baseline: 2.1338x; 2.1338x over previous
import jax
import jax.numpy as jnp
from jax import lax
from jax.experimental import pallas as pl
from jax.experimental.pallas import tpu as pltpu

N = 50000
D = 256
S = 512
R = 2000
NB = N // R
WIN = 72


def _body(x_ref, bseg_ref, bsm_ref, w_ref, bias_ref, out_ref, m_ref, z_ref):
    i = pl.program_id(0)

    @pl.when(i == 0)
    def _init():
        m_ref[0] = -jnp.inf
        z_ref[0] = 0.0
        out_ref[...] = jnp.zeros_like(out_ref)

    xb = x_ref[...]
    s = jnp.dot(xb, w_ref[...], preferred_element_type=jnp.float32)
    s = s + bias_ref[0, 0]

    m_old = m_ref[0]
    m_new = jnp.maximum(m_old, jnp.max(s))
    p = jnp.exp(s - m_new)
    z_ref[0] = z_ref[0] * jnp.exp(m_old - m_new) + jnp.sum(p)
    m_ref[0] = m_new

    @pl.when(jnp.logical_and(i > 0, m_new > m_old))
    def _rescale():
        out_ref[...] = out_ref[...] * jnp.exp(m_old - m_new)

    seg = bseg_ref[0, 0, :]
    xp = (xb * p).astype(jnp.bfloat16)

    base8 = jnp.minimum((bsm_ref[0, 0, 0] // 8) * 8, S - WIN)
    hi = jnp.max(seg)
    in_window = hi - base8 < WIN

    @pl.when(in_window)
    def _fast():
        offs = seg - base8
        onehot = (lax.broadcasted_iota(jnp.int32, (WIN, R), 0) == offs[None, :])
        contrib = jnp.dot(onehot.astype(jnp.bfloat16), xp,
                          preferred_element_type=jnp.float32)
        out_ref[pl.ds(base8, WIN), :] = out_ref[pl.ds(base8, WIN), :] + contrib

    @pl.when(jnp.logical_not(in_window))
    def _slow():
        onehot = (lax.broadcasted_iota(jnp.int32, (S, R), 0) == seg[None, :])
        contrib = jnp.dot(onehot.astype(jnp.bfloat16), xp,
                          preferred_element_type=jnp.float32)
        out_ref[...] = out_ref[...] + contrib

    @pl.when(i == NB - 1)
    def _fin():
        out_ref[...] = out_ref[...] * (1.0 / z_ref[0])


def kernel(x, batch, W, b):
    b3 = batch.astype(jnp.int32).reshape(NB, 1, R)
    return pl.pallas_call(
        _body,
        grid=(NB,),
        in_specs=[
            pl.BlockSpec((R, D), lambda i: (i, 0)),
            pl.BlockSpec((1, 1, R), lambda i: (i, 0, 0)),
            pl.BlockSpec((1, 1, R), lambda i: (i, 0, 0),
                         memory_space=pltpu.SMEM),
            pl.BlockSpec((D, 1), lambda i: (0, 0)),
            pl.BlockSpec((1, 1), lambda i: (0, 0)),
        ],
        out_specs=pl.BlockSpec((S, D), lambda i: (0, 0)),
        out_shape=jax.ShapeDtypeStruct((S, D), jnp.float32),
        scratch_shapes=[pltpu.SMEM((1,), jnp.float32),
                        pltpu.SMEM((1,), jnp.float32)],
    )(x, b3, b3, W, b.reshape(1, 1))
